# baseline (speedup 1.0000x reference)
import functools

import jax
import jax.numpy as jnp
from jax import lax
from jax.experimental import pallas as pl
from jax.experimental.pallas import tpu as pltpu
from jax.experimental.pallas import tpu_sc as plsc


def _make_bcast_kernel(batch, seq, dim):
    info = plsc.get_sparse_core_info()
    nc, ns = info.num_cores, info.num_subcores
    nw = nc * ns
    assert seq % nw == 0
    rows_per_w = seq // nw
    chunk = 16
    while rows_per_w % chunk:
        chunk //= 2
    n_chunks = rows_per_w // chunk
    n_a = (batch + 1) // 2
    n_b = batch - n_a

    mesh = plsc.VectorSubcoreMesh(core_axis_name="c", subcore_axis_name="s")

    @functools.partial(
        pl.kernel,
        mesh=mesh,
        out_type=jax.ShapeDtypeStruct((batch, seq, dim), jnp.float32),
        scratch_types=[
            pltpu.VMEM((chunk, dim), jnp.float32),
            pltpu.VMEM((chunk, dim), jnp.float32),
            pltpu.VMEM_SHARED((ns, chunk, dim), jnp.float32),
            pltpu.VMEM_SHARED((ns, chunk, dim), jnp.float32),
            pltpu.SemaphoreType.DMA,
            pltpu.SemaphoreType.DMA,
            pltpu.SemaphoreType.DMA,
            pltpu.SemaphoreType.DMA,
        ],
    )
    def bcast(table_hbm, out_hbm, tb0, tb1, sb0, sb1, rsem, wsem, srsem, swsem):
        sid = lax.axis_index("s")
        wid = sid * nc + lax.axis_index("c")
        base = wid * rows_per_w
        tbufs = (tb0, tb1)
        sbufs = (sb0, sb1)
        reads_a = [None, None]
        writes_a = [None, None]
        reads_b = [None, None]
        writes_b = [None, None]
        reads_a[0] = pltpu.async_copy(
            table_hbm.at[pl.ds(base, chunk)], tbufs[0], rsem
        )
        if n_b:
            reads_b[0] = pltpu.async_copy(
                table_hbm.at[pl.ds(base, chunk)], sbufs[0].at[sid], srsem
            )
        for i in range(n_chunks):
            cur = i % 2
            nxt = (i + 1) % 2
            row0 = base + i * chunk
            if i + 1 < n_chunks:
                row_n = base + (i + 1) * chunk
                if writes_a[nxt] is not None:
                    for d in writes_a[nxt]:
                        d.wait()
                    writes_a[nxt] = None
                reads_a[nxt] = pltpu.async_copy(
                    table_hbm.at[pl.ds(row_n, chunk)], tbufs[nxt], rsem
                )
                if n_b:
                    if writes_b[nxt] is not None:
                        for d in writes_b[nxt]:
                            d.wait()
                        writes_b[nxt] = None
                    reads_b[nxt] = pltpu.async_copy(
                        table_hbm.at[pl.ds(row_n, chunk)], sbufs[nxt].at[sid], srsem
                    )
            reads_a[cur].wait()
            writes_a[cur] = [
                pltpu.async_copy(
                    tbufs[cur], out_hbm.at[b, pl.ds(row0, chunk)], wsem
                )
                for b in range(n_a)
            ]
            if n_b:
                reads_b[cur].wait()
                writes_b[cur] = [
                    pltpu.async_copy(
                        sbufs[cur].at[sid], out_hbm.at[n_a + b, pl.ds(row0, chunk)], swsem
                    )
                    for b in range(n_b)
                ]
        for pending in writes_a + writes_b:
            if pending is not None:
                for d in pending:
                    d.wait()

    return bcast


def kernel(x, position_embedding):
    batch, seq, dim = x.shape
    fn = _make_bcast_kernel(batch, seq, dim)
    return fn(position_embedding)
